# baseline (speedup 1.0000x reference)
import math

import jax
import jax.numpy as jnp
from jax import lax
from jax.experimental import pallas as pl
from jax.experimental.pallas import tpu as pltpu

_K = 5
_BR = 128
_BC = 1024
_NS = 4


def _mlp_body(x_ref, W1_ref, b1_ref, W2_ref, b2_ref, h_ref):
    h = jnp.dot(x_ref[...], W1_ref[...], preferred_element_type=jnp.float32)
    h = jnp.maximum(h + b1_ref[...], 0.0)
    h = (jnp.dot(h, W2_ref[...], preferred_element_type=jnp.float32)
         + b2_ref[...])
    h_ref[...] = h.T.astype(jnp.bfloat16)


def _dotT(vT, blk):
    return lax.dot_general(vT, blk, (((1,), (1,)), ((), ())),
                           preferred_element_type=jnp.float32)


def _mega_body(adj_hbm, poly_hbm, hT_ref, c_ref,
               y_ref,
               m16, fa, fp, pT, accT,
               sa, sp):
    n = adj_hbm.shape[0]
    nb = n // _BR
    nh = nb // 2
    nc = n // _BC
    ns = _NS // 2

    def a_in(half, b, slot):
        return pltpu.make_async_copy(
            adj_hbm.at[pl.ds((half * nh + b) * _BR, _BR), :],
            fa.at[half * ns + slot], sa.at[half * ns + slot])

    def p_in(half, b, slot):
        return pltpu.make_async_copy(
            poly_hbm.at[pl.ds((half * nh + b) * _BR, _BR), :],
            fp.at[half * ns + slot], sp.at[half * ns + slot])

    hT = hT_ref[...]

    for s in range(ns):
        a_in(0, s, s).start()
        a_in(1, s, s).start()

    def body_a(b, _):
        slot = lax.rem(b, ns)
        for half in (0, 1):
            rows = pl.ds((half * nh + b) * _BR, _BR)
            a_in(half, b, slot).wait()
            ablk = fa[half * ns + slot].astype(jnp.bfloat16)
            m16[rows, :] = ablk

            @pl.when(b + ns < nh)
            def _next(half=half, b=b, slot=slot):
                a_in(half, b + ns, slot).start()

            pT[1, :, rows] = _dotT(hT, ablk).astype(jnp.bfloat16)
        return 0

    lax.fori_loop(0, nh, body_a, 0)

    for s in range(ns):
        p_in(0, s, s).start()
        p_in(1, s, s).start()

    for t in range(2, _K + 1):
        for c in range(nc):
            chunk = pl.ds(c * _BC, _BC)
            pT[t, :, chunk] = _dotT(pT[t - 1], m16[chunk, :]).astype(
                jnp.bfloat16)

    acc0T = (c_ref[_K, 0] * hT.astype(jnp.float32)).astype(jnp.bfloat16)

    def body_p(b, _):
        slot = lax.rem(b, ns)
        for half in (0, 1):
            rows = pl.ds((half * nh + b) * _BR, _BR)
            p_in(half, b, slot).wait()
            pblk = fp[half * ns + slot].astype(jnp.bfloat16)
            m16[rows, :] = pblk

            @pl.when(b + ns < nh)
            def _next(half=half, b=b, slot=slot):
                p_in(half, b + ns, slot).start()

            acc1T = (_dotT(acc0T, pblk)
                     + c_ref[_K - 1, 0] * pT[1, :, rows].astype(jnp.float32))
            accT[1, :, rows] = acc1T.astype(jnp.bfloat16)
        return 0

    lax.fori_loop(0, nh, body_p, 0)

    for t in range(2, _K + 1):
        cur = (t - 1) % 2
        nxt = t % 2
        last = t == _K
        for c in range(nc):
            chunk = pl.ds(c * _BC, _BC)
            accnT = (_dotT(accT[cur], m16[chunk, :])
                     + c_ref[_K - t, 0] * pT[t, :, chunk].astype(jnp.float32))
            if not last:
                accT[nxt, :, chunk] = accnT.astype(jnp.bfloat16)
            else:
                m = jnp.max(accnT, axis=0, keepdims=True)
                lse = (jnp.log(jnp.sum(jnp.exp(accnT - m), axis=0,
                                       keepdims=True)) + m)
                y_ref[chunk, :] = (accnT - lse).T


def kernel(x, adj, poly_item, W1, b1, W2, b2, filter_param):
    N, D_IN = x.shape
    D_HID = W1.shape[1]
    D_OUT = W2.shape[1]

    fp = jax.nn.relu(filter_param[:, 0])
    binom = jnp.asarray([math.comb(_K, i) / 2.0 ** _K for i in range(_K + 1)],
                        jnp.float32)
    coefs = jnp.zeros((8, 1), jnp.float32).at[:_K + 1, 0].set(binom * fp)

    BM = 256
    hT = pl.pallas_call(
        _mlp_body,
        grid=(N // BM,),
        in_specs=[
            pl.BlockSpec((BM, D_IN), lambda i: (i, 0)),
            pl.BlockSpec((D_IN, D_HID), lambda i: (0, 0)),
            pl.BlockSpec((1, D_HID), lambda i: (0, 0)),
            pl.BlockSpec((D_HID, D_OUT), lambda i: (0, 0)),
            pl.BlockSpec((1, D_OUT), lambda i: (0, 0)),
        ],
        out_specs=pl.BlockSpec((D_OUT, BM), lambda i: (0, i)),
        out_shape=jax.ShapeDtypeStruct((D_OUT, N), jnp.bfloat16),
    )(x, W1, b1.reshape(1, -1), W2, b2.reshape(1, -1))

    y = pl.pallas_call(
        _mega_body,
        in_specs=[
            pl.BlockSpec(memory_space=pl.ANY),
            pl.BlockSpec(memory_space=pl.ANY),
            pl.BlockSpec(memory_space=pltpu.VMEM),
            pl.BlockSpec(memory_space=pltpu.SMEM),
        ],
        out_specs=pl.BlockSpec(memory_space=pltpu.VMEM),
        out_shape=jax.ShapeDtypeStruct((N, D_OUT), jnp.float32),
        scratch_shapes=[
            pltpu.VMEM((N, N), jnp.bfloat16),
            pltpu.VMEM((_NS, _BR, N), jnp.float32),
            pltpu.VMEM((_NS, _BR, N), jnp.float32),
            pltpu.VMEM((_K + 1, D_OUT, N), jnp.bfloat16),
            pltpu.VMEM((2, D_OUT, N), jnp.bfloat16),
            pltpu.SemaphoreType.DMA((_NS,)),
            pltpu.SemaphoreType.DMA((_NS,)),
        ],
        compiler_params=pltpu.CompilerParams(
            vmem_limit_bytes=64 * 1024 * 1024),
    )(adj, poly_item, hT, coefs)
    return y


# speedup vs baseline: 1.0537x; 1.0537x over previous
import math

import jax
import jax.numpy as jnp
from jax import lax
from jax.experimental import pallas as pl
from jax.experimental.pallas import tpu as pltpu

_K = 5
_BR = 128
_BC = 1024
_NS = 4


def _mlp_body(x_ref, W1_ref, b1_ref, W2_ref, b2_ref, h_ref):
    h = jnp.dot(x_ref[...], W1_ref[...], preferred_element_type=jnp.float32)
    h = jnp.maximum(h + b1_ref[...], 0.0)
    h = (jnp.dot(h, W2_ref[...], preferred_element_type=jnp.float32)
         + b2_ref[...])
    h_ref[...] = h.T.astype(jnp.bfloat16)


def _dotT(vT, blk):
    return lax.dot_general(vT, blk, (((1,), (1,)), ((), ())),
                           preferred_element_type=jnp.float32)


def _mega_body(adj_hbm, poly_hbm, hT_ref, c_ref,
               y_ref,
               m16, fa, fp, pT, accT,
               sa, sp):
    n = adj_hbm.shape[0]
    nb = n // _BR
    nc = n // _BC

    def a_in(b, slot):
        return pltpu.make_async_copy(
            adj_hbm.at[pl.ds(b * _BR, _BR), :], fa.at[slot], sa.at[slot])

    def p_in(b, slot):
        return pltpu.make_async_copy(
            poly_hbm.at[pl.ds(b * _BR, _BR), :], fp.at[slot], sp.at[slot])

    hT = hT_ref[...]

    for s in range(_NS):
        a_in(s, s).start()

    def body_a(b, _):
        slot = lax.rem(b, _NS)
        rows = pl.ds(b * _BR, _BR)
        a_in(b, slot).wait()
        ablk = fa[slot].astype(jnp.bfloat16)
        m16[rows, :] = ablk

        @pl.when(b + _NS < nb)
        def _next():
            a_in(b + _NS, slot).start()

        pT[1, :, rows] = _dotT(hT, ablk).astype(jnp.bfloat16)
        return 0

    lax.fori_loop(0, nb, body_a, 0)

    for s in range(_NS):
        p_in(s, s).start()

    for t in range(2, _K + 1):
        for c in range(nc):
            chunk = pl.ds(c * _BC, _BC)
            pT[t, :, chunk] = _dotT(pT[t - 1], m16[chunk, :]).astype(
                jnp.bfloat16)

    acc0T = (c_ref[_K, 0] * hT.astype(jnp.float32)).astype(jnp.bfloat16)

    def body_p(b, _):
        slot = lax.rem(b, _NS)
        rows = pl.ds(b * _BR, _BR)
        p_in(b, slot).wait()
        pblk = fp[slot].astype(jnp.bfloat16)
        m16[rows, :] = pblk

        @pl.when(b + _NS < nb)
        def _next():
            p_in(b + _NS, slot).start()

        acc1T = (_dotT(acc0T, pblk)
                 + c_ref[_K - 1, 0] * pT[1, :, rows].astype(jnp.float32))
        accT[1, :, rows] = acc1T.astype(jnp.bfloat16)
        return 0

    lax.fori_loop(0, nb, body_p, 0)

    for t in range(2, _K + 1):
        cur = (t - 1) % 2
        nxt = t % 2
        last = t == _K
        for c in range(nc):
            chunk = pl.ds(c * _BC, _BC)
            accnT = (_dotT(accT[cur], m16[chunk, :])
                     + c_ref[_K - t, 0] * pT[t, :, chunk].astype(jnp.float32))
            if not last:
                accT[nxt, :, chunk] = accnT.astype(jnp.bfloat16)
            else:
                m = jnp.max(accnT, axis=0, keepdims=True)
                lse = (jnp.log(jnp.sum(jnp.exp(accnT - m), axis=0,
                                       keepdims=True)) + m)
                y_ref[chunk, :] = (accnT - lse).T


def kernel(x, adj, poly_item, W1, b1, W2, b2, filter_param):
    N, D_IN = x.shape
    D_HID = W1.shape[1]
    D_OUT = W2.shape[1]

    fp = jax.nn.relu(filter_param[:, 0])
    binom = jnp.asarray([math.comb(_K, i) / 2.0 ** _K for i in range(_K + 1)],
                        jnp.float32)
    coefs = jnp.zeros((8, 1), jnp.float32).at[:_K + 1, 0].set(binom * fp)

    BM = 256
    hT = pl.pallas_call(
        _mlp_body,
        grid=(N // BM,),
        in_specs=[
            pl.BlockSpec((BM, D_IN), lambda i: (i, 0)),
            pl.BlockSpec((D_IN, D_HID), lambda i: (0, 0)),
            pl.BlockSpec((1, D_HID), lambda i: (0, 0)),
            pl.BlockSpec((D_HID, D_OUT), lambda i: (0, 0)),
            pl.BlockSpec((1, D_OUT), lambda i: (0, 0)),
        ],
        out_specs=pl.BlockSpec((D_OUT, BM), lambda i: (0, i)),
        out_shape=jax.ShapeDtypeStruct((D_OUT, N), jnp.bfloat16),
    )(x, W1, b1.reshape(1, -1), W2, b2.reshape(1, -1))

    y = pl.pallas_call(
        _mega_body,
        in_specs=[
            pl.BlockSpec(memory_space=pl.ANY),
            pl.BlockSpec(memory_space=pl.ANY),
            pl.BlockSpec(memory_space=pltpu.VMEM),
            pl.BlockSpec(memory_space=pltpu.SMEM),
        ],
        out_specs=pl.BlockSpec(memory_space=pltpu.VMEM),
        out_shape=jax.ShapeDtypeStruct((N, D_OUT), jnp.float32),
        scratch_shapes=[
            pltpu.VMEM((N, N), jnp.bfloat16),
            pltpu.VMEM((_NS, _BR, N), jnp.float32),
            pltpu.VMEM((_NS, _BR, N), jnp.float32),
            pltpu.VMEM((_K + 1, D_OUT, N), jnp.bfloat16),
            pltpu.VMEM((2, D_OUT, N), jnp.bfloat16),
            pltpu.SemaphoreType.DMA((_NS,)),
            pltpu.SemaphoreType.DMA((_NS,)),
        ],
        compiler_params=pltpu.CompilerParams(
            vmem_limit_bytes=64 * 1024 * 1024),
    )(adj, poly_item, hT, coefs)
    return y
